# baseline (speedup 1.0000x reference)
import functools

import jax
import jax.numpy as jnp
from jax import lax
from jax.experimental import pallas as pl
from jax.experimental.pallas import tpu as pltpu
from jax.experimental.pallas import tpu_sc as plsc

N_U = 10000
N_I = 10000
FDIM = 128
NS = 16
NC = 2
EB = 128
SB = 128
RD = 5


def _dense_cell(x, Wt, bt, Wi, bi):
    n = x.shape[0]
    bn = 2000
    nb = n // bn

    def body(x_ref, wt_ref, bt_ref, wi_ref, bi_ref, g_ref):
        h = pl.program_id(1)
        xv = x_ref[...]
        h1 = jnp.dot(xv, wt_ref[...], preferred_element_type=jnp.float32) + bt_ref[...]
        h12 = h1 + jnp.dot(xv * xv, wi_ref[...], preferred_element_type=jnp.float32) + bi_ref[...]
        g_ref[...] = jnp.where(h == 0, h12[:, :64], h12[:, 64:])

    return pl.pallas_call(
        body,
        grid=(nb, 2),
        in_specs=[
            pl.BlockSpec((bn, FDIM), lambda i, h: (i, 0)),
            pl.BlockSpec((FDIM, FDIM), lambda i, h: (0, 0)),
            pl.BlockSpec((1, FDIM), lambda i, h: (0, 0)),
            pl.BlockSpec((FDIM, FDIM), lambda i, h: (0, 0)),
            pl.BlockSpec((1, FDIM), lambda i, h: (0, 0)),
        ],
        out_specs=pl.BlockSpec((bn, 64), lambda i, h: (h * nb + i, 0)),
        out_shape=jax.ShapeDtypeStruct((2 * n, 64), jnp.float32),
    )(x, Wt, bt.reshape(1, FDIM), Wi, bi.reshape(1, FDIM))


@functools.lru_cache(maxsize=None)
def _make_spmm(n, n_chunks):
    e_per_tile = n_chunks * EB
    dump_rows = 200
    dump_chunks = n // dump_rows
    dump_iters = (dump_chunks + NS - 1) // NS
    mesh = plsc.VectorSubcoreMesh(
        core_axis_name="c", subcore_axis_name="s", num_cores=NC, num_subcores=NS)

    @functools.partial(
        pl.kernel,
        out_type=jax.ShapeDtypeStruct((2 * n, 64), jnp.float32),
        mesh=mesh,
        scratch_types=(
            [pltpu.VMEM((3, EB), jnp.int32)] * RD
            + [pltpu.VMEM((EB // SB, SB), jnp.int32)] * RD
            + [pltpu.VMEM((EB, 64), jnp.float32)] * RD
            + [pltpu.VMEM_SHARED((n, 64), jnp.float32)]
            + [pltpu.SemaphoreType.DMA] * (3 * RD)
        ),
        compiler_params=pltpu.CompilerParams(
            use_tc_tiling_on_sc=False, needs_layout_passes=False),
    )
    def spmm(g_hbm, edata_hbm, out_hbm, *scr):
        ibuf = scr[0:RD]
        dbuf = scr[RD:2 * RD]
        rows = scr[2 * RD:3 * RD]
        acc = scr[3 * RD]
        sem_i = scr[3 * RD + 1:4 * RD + 1]
        sem_g = scr[4 * RD + 1:5 * RD + 1]
        sem_c = scr[5 * RD + 1:6 * RD + 1]
        c = lax.axis_index("c")
        s = lax.axis_index("s")
        cn = c * n
        base = s * e_per_tile

        def idx_desc(t, b):
            tc = jnp.minimum(t, n_chunks - 1)
            return pltpu.make_async_copy(
                edata_hbm.at[:, pl.ds(base + tc * EB, EB)], ibuf[b], sem_i[b])

        def gather_start(b):
            for h in range(EB // SB):
                pltpu.async_copy(
                    g_hbm.at[ibuf[b].at[0, pl.ds(SB * h, SB)]],
                    rows[b].at[pl.ds(SB * h, SB)], sem_g[b])

        def gather_wait(b):
            for h in range(EB // SB):
                pltpu.make_async_copy(
                    g_hbm.at[ibuf[b].at[0, pl.ds(SB * h, SB)]],
                    rows[b].at[pl.ds(SB * h, SB)], sem_g[b]).wait()

        def scatter_start(b):
            for h in range(EB // SB):
                pltpu.async_copy(
                    rows[b].at[pl.ds(SB * h, SB)],
                    acc.at[dbuf[b].at[h]], sem_c[b], add=True)

        def scatter_wait(b):
            for h in range(EB // SB):
                pltpu.make_async_copy(
                    rows[b].at[pl.ds(SB * h, SB)],
                    acc.at[dbuf[b].at[h]], sem_c[b]).wait()

        def adjust(b):
            for k in range(EB // 16):
                sl = pl.ds(16 * k, 16)
                ibuf[b][0, sl] = ibuf[b][0, sl] + cn

        def scale(b):
            @plsc.parallel_loop(0, EB // 16, unroll=2)
            def _(g):
                wvec = plsc.bitcast(ibuf[b][2, pl.ds(16 * g, 16)], jnp.float32)
                for jj in range(16):
                    wj = wvec[jj]
                    j = 16 * g + jj
                    for k in range(4):
                        sl = pl.ds(16 * k, 16)
                        rows[b][j, sl] = rows[b][j, sl] * wj

        def zb(j, carry):
            for k in range(4):
                rows[0][j, pl.ds(16 * k, 16)] = jnp.zeros((16,), jnp.float32)
            return carry
        lax.fori_loop(0, EB, zb, 0)
        full_init = n // EB
        tail_init = n - full_init * EB
        for t in range((full_init + NS - 1) // NS):
            q = s + NS * t
            @pl.when(q < full_init)
            def _():
                pltpu.sync_copy(rows[0], acc.at[pl.ds(q * EB, EB)])
        if tail_init:
            @pl.when(s == 0)
            def _():
                pltpu.sync_copy(rows[0].at[pl.ds(0, tail_init)],
                                acc.at[pl.ds(full_init * EB, tail_init)])
        plsc.subcore_barrier()

        for b in range(RD):
            idx_desc(b, b).start()
        for b in range(2):
            idx_desc(b, b).wait()
            adjust(b)
            gather_start(b)

        def step(t, b):
            bp2 = (b + 2) % RD
            idx_desc(t + 2, bp2).wait()
            adjust(bp2)
            @pl.when(t > RD - 3)
            def _():
                scatter_wait(bp2)
            gather_start(bp2)
            gather_wait(b)
            scale(b)
            for h in range(EB // SB):
                for k in range(SB // 16):
                    dbuf[b][h, pl.ds(16 * k, 16)] = ibuf[b][1, pl.ds(SB * h + 16 * k, 16)]
            scatter_start(b)
            idx_desc(t + RD, b).start()

        def body(tt, carry):
            for b in range(RD):
                step(RD * tt + b, b)
            return carry
        lax.fori_loop(0, n_chunks // RD, body, 0)

        nc = n_chunks
        for t in range(nc + 2, nc + RD):
            idx_desc(t, t % RD).wait()
        gather_wait(nc % RD)
        gather_wait((nc + 1) % RD)
        for t in range(nc + 2 - RD, nc):
            scatter_wait(t % RD)
        plsc.subcore_barrier()

        for t in range(dump_iters):
            q = s + NS * t
            @pl.when(q < dump_chunks)
            def _():
                lo = q * dump_rows
                pltpu.sync_copy(acc.at[pl.ds(lo, dump_rows)],
                                out_hbm.at[pl.ds(cn + lo, dump_rows)])

    return spmm


def _prep_edges(edge_index, edge_weight, e_pad, n):
    e = edge_index.shape[1]
    pad = e_pad - e
    pad_idx = jnp.arange(pad, dtype=jnp.int32) % n
    src = jnp.concatenate([edge_index[1], pad_idx])
    dst = jnp.concatenate([edge_index[0], pad_idx])
    w = jnp.concatenate([edge_weight, jnp.zeros((pad,), jnp.float32)])
    return jnp.stack([src, dst, jax.lax.bitcast_convert_type(w, jnp.int32)])


def _attention(S_e, b0e, b1e, S_ui, b0ui, b1ui, x, Wte, bte, Wtui, btui, W1, b1, W2):
    n = x.shape[0]
    bn = 2000
    nb = n // bn

    def body(se0, se1, su0, su1, x_ref, wte, bter, wtui, btuir, w1, b1r, w2, out_ref):
        xv = x_ref[...]
        z0 = (jnp.concatenate([se0[...], se1[...]], axis=1)
              + jnp.dot(xv, wte[...], preferred_element_type=jnp.float32) + bter[...])
        z1 = (jnp.concatenate([su0[...], su1[...]], axis=1)
              + jnp.dot(xv, wtui[...], preferred_element_type=jnp.float32) + btuir[...])
        t0 = jnp.tanh(jnp.dot(z0, w1[...], preferred_element_type=jnp.float32) + b1r[...])
        t1 = jnp.tanh(jnp.dot(z1, w1[...], preferred_element_type=jnp.float32) + b1r[...])
        s0 = jnp.sum(t0 * w2[...], axis=1, keepdims=True)
        s1 = jnp.sum(t1 * w2[...], axis=1, keepdims=True)
        m = jnp.maximum(s0, s1)
        e0 = jnp.exp(s0 - m)
        e1 = jnp.exp(s1 - m)
        out_ref[...] = (e0 * z0 + e1 * z1) / (e0 + e1)

    return pl.pallas_call(
        body,
        grid=(nb,),
        in_specs=[
            pl.BlockSpec((bn, 64), lambda i, b=b0e // bn: (b + i, 0)),
            pl.BlockSpec((bn, 64), lambda i, b=b1e // bn: (b + i, 0)),
            pl.BlockSpec((bn, 64), lambda i, b=b0ui // bn: (b + i, 0)),
            pl.BlockSpec((bn, 64), lambda i, b=b1ui // bn: (b + i, 0)),
            pl.BlockSpec((bn, FDIM), lambda i: (i, 0)),
            pl.BlockSpec((FDIM, FDIM), lambda i: (0, 0)),
            pl.BlockSpec((1, FDIM), lambda i: (0, 0)),
            pl.BlockSpec((FDIM, FDIM), lambda i: (0, 0)),
            pl.BlockSpec((1, FDIM), lambda i: (0, 0)),
            pl.BlockSpec((FDIM, 32), lambda i: (0, 0)),
            pl.BlockSpec((1, 32), lambda i: (0, 0)),
            pl.BlockSpec((1, 32), lambda i: (0, 0)),
        ],
        out_specs=pl.BlockSpec((bn, FDIM), lambda i: (i, 0)),
        out_shape=jax.ShapeDtypeStruct((n, FDIM), jnp.float32),
    )(S_e, S_e, S_ui, S_ui, x, Wte, bte.reshape(1, FDIM), Wtui,
      btui.reshape(1, FDIM), W1, b1.reshape(1, 32), W2.reshape(1, 32))


def kernel(u_feature, i_feature, u2i_edge_index, u2i_edge_weight,
           u2e_edge_index, u2e_edge_weight, i2e_edge_index, i2e_edge_weight,
           u2e_Wt, u2e_bt, u2e_Wi, u2e_bi,
           i2e_Wt, i2e_bt, i2e_Wi, i2e_bi,
           u2i_Wt, u2i_bt, u2i_Wi, u2i_bi,
           uatt_W1, uatt_b1, uatt_W2,
           iatt_W1, iatt_b1, iatt_W2):
    n_ui = N_U + N_I
    feats = jnp.concatenate([u_feature, i_feature], axis=0)

    g_e = _dense_cell(u_feature, u2e_Wt, u2e_bt, u2e_Wi, u2e_bi)
    g_i = _dense_cell(i_feature, i2e_Wt, i2e_bt, i2e_Wi, i2e_bi)
    g_ui = _dense_cell(feats, u2i_Wt, u2i_bt, u2i_Wi, u2i_bi)

    nch_small = -(-160000 // (NS * EB * RD)) * RD
    nch_big = -(-320000 // (NS * EB * RD)) * RD
    e_small = NS * nch_small * EB
    e_big = NS * nch_big * EB
    ed_e = _prep_edges(u2e_edge_index, u2e_edge_weight, e_small, N_U)
    ed_i = _prep_edges(i2e_edge_index, i2e_edge_weight, e_small, N_I)
    ed_ui = _prep_edges(u2i_edge_index, u2i_edge_weight, e_big, n_ui)

    spmm_small = _make_spmm(N_U, nch_small)
    spmm_big = _make_spmm(n_ui, nch_big)

    def _tie(edata, v):
        return edata + jax.lax.convert_element_type(v * 0.0, jnp.int32)

    s_e = spmm_small(g_e, ed_e)
    s_i = spmm_small(g_i, _tie(ed_i, s_e[0, 0]))
    s_ui = spmm_big(g_ui, _tie(ed_ui, s_i[0, 0]))

    u_out = _attention(s_e, 0, N_U, s_ui, 0, n_ui, u_feature,
                       u2e_Wt, u2e_bt, u2i_Wt, u2i_bt, uatt_W1, uatt_b1, uatt_W2)
    i_out = _attention(s_i, 0, N_I, s_ui, N_U, n_ui + N_U, i_feature,
                       i2e_Wt, i2e_bt, u2i_Wt, u2i_bt, iatt_W1, iatt_b1, iatt_W2)
    return (u_out, i_out)


# speedup vs baseline: 1.1059x; 1.1059x over previous
import functools

import jax
import jax.numpy as jnp
from jax import lax
from jax.experimental import pallas as pl
from jax.experimental.pallas import tpu as pltpu
from jax.experimental.pallas import tpu_sc as plsc

N_U = 10000
N_I = 10000
FDIM = 128
NS = 16
NC = 2
EB = 128
SB = 128
RD = 4


def _dense_cell(x, Wt, bt, Wi, bi):
    n = x.shape[0]
    bn = 2000
    nb = n // bn

    def body(x_ref, wt_ref, bt_ref, wi_ref, bi_ref, g_ref):
        h = pl.program_id(1)
        xv = x_ref[...]
        h1 = jnp.dot(xv, wt_ref[...], preferred_element_type=jnp.float32) + bt_ref[...]
        h12 = h1 + jnp.dot(xv * xv, wi_ref[...], preferred_element_type=jnp.float32) + bi_ref[...]
        g_ref[...] = jnp.where(h == 0, h12[:, :64], h12[:, 64:])

    return pl.pallas_call(
        body,
        grid=(nb, 2),
        in_specs=[
            pl.BlockSpec((bn, FDIM), lambda i, h: (i, 0)),
            pl.BlockSpec((FDIM, FDIM), lambda i, h: (0, 0)),
            pl.BlockSpec((1, FDIM), lambda i, h: (0, 0)),
            pl.BlockSpec((FDIM, FDIM), lambda i, h: (0, 0)),
            pl.BlockSpec((1, FDIM), lambda i, h: (0, 0)),
        ],
        out_specs=pl.BlockSpec((bn, 64), lambda i, h: (h * nb + i, 0)),
        out_shape=jax.ShapeDtypeStruct((2 * n, 64), jnp.float32),
    )(x, Wt, bt.reshape(1, FDIM), Wi, bi.reshape(1, FDIM))


@functools.lru_cache(maxsize=None)
def _make_spmm(n, n_chunks):
    e_per_tile = n_chunks * EB
    dump_rows = 200
    dump_chunks = n // dump_rows
    dump_iters = (dump_chunks + NS - 1) // NS
    mesh = plsc.VectorSubcoreMesh(
        core_axis_name="c", subcore_axis_name="s", num_cores=NC, num_subcores=NS)

    @functools.partial(
        pl.kernel,
        out_type=jax.ShapeDtypeStruct((2 * n, 64), jnp.float32),
        mesh=mesh,
        scratch_types=(
            [pltpu.VMEM((3, EB), jnp.int32)] * RD
            + [pltpu.VMEM((EB // SB, SB), jnp.int32)] * RD
            + [pltpu.VMEM((EB, 64), jnp.float32)] * RD
            + [pltpu.VMEM_SHARED((n, 64), jnp.float32)]
            + [pltpu.SemaphoreType.DMA] * (3 * RD)
        ),
        compiler_params=pltpu.CompilerParams(
            use_tc_tiling_on_sc=False, needs_layout_passes=False),
    )
    def spmm(g_hbm, edata_hbm, out_hbm, *scr):
        ibuf = scr[0:RD]
        dbuf = scr[RD:2 * RD]
        rows = scr[2 * RD:3 * RD]
        acc = scr[3 * RD]
        sem_i = scr[3 * RD + 1:4 * RD + 1]
        sem_g = scr[4 * RD + 1:5 * RD + 1]
        sem_c = scr[5 * RD + 1:6 * RD + 1]
        c = lax.axis_index("c")
        s = lax.axis_index("s")
        cn = c * n
        base = s * e_per_tile

        def idx_desc(t, b):
            tc = jnp.minimum(t, n_chunks - 1)
            return pltpu.make_async_copy(
                edata_hbm.at[:, pl.ds(base + tc * EB, EB)], ibuf[b], sem_i[b])

        def gather_start(b):
            for h in range(EB // SB):
                pltpu.async_copy(
                    g_hbm.at[ibuf[b].at[0, pl.ds(SB * h, SB)]],
                    rows[b].at[pl.ds(SB * h, SB)], sem_g[b])

        def gather_wait(b):
            for h in range(EB // SB):
                pltpu.make_async_copy(
                    g_hbm.at[ibuf[b].at[0, pl.ds(SB * h, SB)]],
                    rows[b].at[pl.ds(SB * h, SB)], sem_g[b]).wait()

        def scatter_start(b):
            for h in range(EB // SB):
                pltpu.async_copy(
                    rows[b].at[pl.ds(SB * h, SB)],
                    acc.at[dbuf[b].at[h]], sem_c[b], add=True)

        def scatter_wait(b):
            for h in range(EB // SB):
                pltpu.make_async_copy(
                    rows[b].at[pl.ds(SB * h, SB)],
                    acc.at[dbuf[b].at[h]], sem_c[b]).wait()

        def adjust(b):
            for k in range(EB // 16):
                sl = pl.ds(16 * k, 16)
                ibuf[b][0, sl] = ibuf[b][0, sl] + cn

        def scale(b):
            @plsc.parallel_loop(0, EB // 16, unroll=2)
            def _(g):
                wvec = plsc.bitcast(ibuf[b][2, pl.ds(16 * g, 16)], jnp.float32)
                for jj in range(16):
                    wj = wvec[jj]
                    j = 16 * g + jj
                    for k in range(4):
                        sl = pl.ds(16 * k, 16)
                        rows[b][j, sl] = rows[b][j, sl] * wj

        def zb(j, carry):
            for k in range(4):
                rows[0][j, pl.ds(16 * k, 16)] = jnp.zeros((16,), jnp.float32)
            return carry
        lax.fori_loop(0, EB, zb, 0)
        full_init = n // EB
        tail_init = n - full_init * EB
        for t in range((full_init + NS - 1) // NS):
            q = s + NS * t
            @pl.when(q < full_init)
            def _():
                pltpu.sync_copy(rows[0], acc.at[pl.ds(q * EB, EB)])
        if tail_init:
            @pl.when(s == 0)
            def _():
                pltpu.sync_copy(rows[0].at[pl.ds(0, tail_init)],
                                acc.at[pl.ds(full_init * EB, tail_init)])
        plsc.subcore_barrier()

        for b in range(RD):
            idx_desc(b, b).start()
        for b in range(2):
            idx_desc(b, b).wait()
            adjust(b)
            gather_start(b)

        def step(t, b):
            bp2 = (b + 2) % RD
            idx_desc(t + 2, bp2).wait()
            adjust(bp2)
            @pl.when(t > RD - 3)
            def _():
                scatter_wait(bp2)
            gather_start(bp2)
            gather_wait(b)
            scale(b)
            for h in range(EB // SB):
                for k in range(SB // 16):
                    dbuf[b][h, pl.ds(16 * k, 16)] = ibuf[b][1, pl.ds(SB * h + 16 * k, 16)]
            scatter_start(b)
            idx_desc(t + RD, b).start()

        def body(tt, carry):
            for b in range(RD):
                step(RD * tt + b, b)
            return carry
        lax.fori_loop(0, n_chunks // RD, body, 0)

        nc = n_chunks
        for t in range(nc + 2, nc + RD):
            idx_desc(t, t % RD).wait()
        gather_wait(nc % RD)
        gather_wait((nc + 1) % RD)
        for t in range(nc + 2 - RD, nc):
            scatter_wait(t % RD)
        plsc.subcore_barrier()

        for t in range(dump_iters):
            q = s + NS * t
            @pl.when(q < dump_chunks)
            def _():
                lo = q * dump_rows
                pltpu.sync_copy(acc.at[pl.ds(lo, dump_rows)],
                                out_hbm.at[pl.ds(cn + lo, dump_rows)])

    return spmm


def _prep_edges(edge_index, edge_weight, e_pad, n):
    e = edge_index.shape[1]
    pad = e_pad - e
    pad_idx = jnp.arange(pad, dtype=jnp.int32) % n
    src = jnp.concatenate([edge_index[1], pad_idx])
    dst = jnp.concatenate([edge_index[0], pad_idx])
    w = jnp.concatenate([edge_weight, jnp.zeros((pad,), jnp.float32)])
    return jnp.stack([src, dst, jax.lax.bitcast_convert_type(w, jnp.int32)])


def _attention(S_e, b0e, b1e, S_ui, b0ui, b1ui, x, Wte, bte, Wtui, btui, W1, b1, W2):
    n = x.shape[0]
    bn = 2000
    nb = n // bn

    def body(se0, se1, su0, su1, x_ref, wte, bter, wtui, btuir, w1, b1r, w2, out_ref):
        xv = x_ref[...]
        z0 = (jnp.concatenate([se0[...], se1[...]], axis=1)
              + jnp.dot(xv, wte[...], preferred_element_type=jnp.float32) + bter[...])
        z1 = (jnp.concatenate([su0[...], su1[...]], axis=1)
              + jnp.dot(xv, wtui[...], preferred_element_type=jnp.float32) + btuir[...])
        t0 = jnp.tanh(jnp.dot(z0, w1[...], preferred_element_type=jnp.float32) + b1r[...])
        t1 = jnp.tanh(jnp.dot(z1, w1[...], preferred_element_type=jnp.float32) + b1r[...])
        s0 = jnp.sum(t0 * w2[...], axis=1, keepdims=True)
        s1 = jnp.sum(t1 * w2[...], axis=1, keepdims=True)
        m = jnp.maximum(s0, s1)
        e0 = jnp.exp(s0 - m)
        e1 = jnp.exp(s1 - m)
        out_ref[...] = (e0 * z0 + e1 * z1) / (e0 + e1)

    return pl.pallas_call(
        body,
        grid=(nb,),
        in_specs=[
            pl.BlockSpec((bn, 64), lambda i, b=b0e // bn: (b + i, 0)),
            pl.BlockSpec((bn, 64), lambda i, b=b1e // bn: (b + i, 0)),
            pl.BlockSpec((bn, 64), lambda i, b=b0ui // bn: (b + i, 0)),
            pl.BlockSpec((bn, 64), lambda i, b=b1ui // bn: (b + i, 0)),
            pl.BlockSpec((bn, FDIM), lambda i: (i, 0)),
            pl.BlockSpec((FDIM, FDIM), lambda i: (0, 0)),
            pl.BlockSpec((1, FDIM), lambda i: (0, 0)),
            pl.BlockSpec((FDIM, FDIM), lambda i: (0, 0)),
            pl.BlockSpec((1, FDIM), lambda i: (0, 0)),
            pl.BlockSpec((FDIM, 32), lambda i: (0, 0)),
            pl.BlockSpec((1, 32), lambda i: (0, 0)),
            pl.BlockSpec((1, 32), lambda i: (0, 0)),
        ],
        out_specs=pl.BlockSpec((bn, FDIM), lambda i: (i, 0)),
        out_shape=jax.ShapeDtypeStruct((n, FDIM), jnp.float32),
    )(S_e, S_e, S_ui, S_ui, x, Wte, bte.reshape(1, FDIM), Wtui,
      btui.reshape(1, FDIM), W1, b1.reshape(1, 32), W2.reshape(1, 32))


def kernel(u_feature, i_feature, u2i_edge_index, u2i_edge_weight,
           u2e_edge_index, u2e_edge_weight, i2e_edge_index, i2e_edge_weight,
           u2e_Wt, u2e_bt, u2e_Wi, u2e_bi,
           i2e_Wt, i2e_bt, i2e_Wi, i2e_bi,
           u2i_Wt, u2i_bt, u2i_Wi, u2i_bi,
           uatt_W1, uatt_b1, uatt_W2,
           iatt_W1, iatt_b1, iatt_W2):
    n_ui = N_U + N_I
    feats = jnp.concatenate([u_feature, i_feature], axis=0)

    g_e = _dense_cell(u_feature, u2e_Wt, u2e_bt, u2e_Wi, u2e_bi)
    g_i = _dense_cell(i_feature, i2e_Wt, i2e_bt, i2e_Wi, i2e_bi)
    g_ui = _dense_cell(feats, u2i_Wt, u2i_bt, u2i_Wi, u2i_bi)

    nch_small = -(-160000 // (NS * EB * RD)) * RD
    nch_big = -(-320000 // (NS * EB * RD)) * RD
    e_small = NS * nch_small * EB
    e_big = NS * nch_big * EB
    ed_e = _prep_edges(u2e_edge_index, u2e_edge_weight, e_small, N_U)
    ed_i = _prep_edges(i2e_edge_index, i2e_edge_weight, e_small, N_I)
    ed_ui = _prep_edges(u2i_edge_index, u2i_edge_weight, e_big, n_ui)

    spmm_small = _make_spmm(N_U, nch_small)
    spmm_big = _make_spmm(n_ui, nch_big)

    s_e = spmm_small(g_e, ed_e)
    s_i = spmm_small(g_i, ed_i)
    s_ui = spmm_big(g_ui, ed_ui)

    u_out = _attention(s_e, 0, N_U, s_ui, 0, n_ui, u_feature,
                       u2e_Wt, u2e_bt, u2i_Wt, u2i_bt, uatt_W1, uatt_b1, uatt_W2)
    i_out = _attention(s_i, 0, N_I, s_ui, N_U, n_ui + N_U, i_feature,
                       i2e_Wt, i2e_bt, u2i_Wt, u2i_bt, iatt_W1, iatt_b1, iatt_W2)
    return (u_out, i_out)
